# baseline (speedup 1.0000x reference)
import functools

import jax
import jax.numpy as jnp
from jax.experimental import pallas as pl
from jax.experimental.pallas import tpu as pltpu
from jax.experimental.pallas import tpu_sc as plsc

_KS = (1, 3, 5, 7, 9, 11, 13)
_NK = len(_KS)
_PADR = 8
_PADL = 8
_NEG = -jnp.inf


def _means_body(x_ref, o_ref):
    s = jnp.sum(x_ref[...], axis=(1, 2))
    o_ref[0, 0, :] = s


def _make_sc_router(bdim, cdim, crdim):
    info = plsc.get_sparse_core_info()
    nw = info.num_cores * info.num_subcores
    lanes = info.num_lanes
    ngroups = bdim * cdim // lanes
    gper_b = cdim // lanes
    assert ngroups <= nw and cdim % lanes == 0 and crdim <= lanes
    mesh = plsc.VectorSubcoreMesh(core_axis_name="c", subcore_axis_name="s")

    @functools.partial(
        pl.kernel, mesh=mesh,
        out_type=(jax.ShapeDtypeStruct((_NK * bdim * cdim,), jnp.float32),
                  jax.ShapeDtypeStruct((bdim * cdim,), jnp.int32)),
        scratch_types=[
            pltpu.VMEM((bdim, cdim), jnp.float32),
            pltpu.VMEM((crdim, cdim), jnp.float32),
            pltpu.VMEM((lanes,), jnp.float32),
            pltpu.VMEM((crdim, lanes), jnp.float32),
            pltpu.VMEM((lanes,), jnp.float32),
            pltpu.VMEM((_NK, lanes), jnp.float32),
            pltpu.VMEM((lanes,), jnp.int32),
        ],
    )
    def sc_router(content_h, w1_h, b1_h, w2t_h, b2_h, alpha_h, nlev_h,
                  content_v, w1_v, b1_v, w2t_v, b2_v, av, nv):
        wid = (jax.lax.axis_index("s") * info.num_cores
               + jax.lax.axis_index("c"))

        @pl.when(wid < ngroups)
        def _():
            co = (wid % gper_b) * lanes
            flat = wid * lanes
            pltpu.sync_copy(content_h, content_v)
            pltpu.sync_copy(w1_h, w1_v)
            pltpu.sync_copy(b1_h, b1_v)
            for r in range(crdim):
                pltpu.sync_copy(w2t_h.at[pl.ds(r * cdim + co, lanes)],
                                w2t_v.at[r])
            pltpu.sync_copy(b2_h.at[pl.ds(co, lanes)], b2_v)

            def shuf(v, idx):
                return jax.lax.gather(
                    v, idx[:, None],
                    jax.lax.GatherDimensionNumbers(
                        offset_dims=(), collapsed_slice_dims=(0,),
                        start_index_map=(0,)),
                    (1,), mode=jax.lax.GatherScatterMode.PROMISE_IN_BOUNDS)

            def bcast(v, i):
                return shuf(v, jnp.full((lanes,), i, jnp.int32))

            lane = jax.lax.iota(jnp.int32, lanes)

            def allsum(v):
                for s in (1, 2, 4, 8):
                    v = v + shuf(v, lane ^ s)
                return v

            vwid = jnp.zeros((lanes,), jnp.int32) + wid
            m0 = jnp.minimum(jnp.maximum(gper_b - vwid, 0),
                             1).astype(jnp.float32)
            m1 = 1.0 - m0
            b1v = b1_v[...]
            glob = b2_v[...]
            for r in range(crdim):
                s0 = jnp.zeros((lanes,), jnp.float32)
                s1 = jnp.zeros((lanes,), jnp.float32)
                for t in range(cdim // lanes):
                    w1c = w1_v[r, pl.ds(t * lanes, lanes)]
                    s0 = s0 + content_v[0, pl.ds(t * lanes, lanes)] * w1c
                    s1 = s1 + content_v[1, pl.ds(t * lanes, lanes)] * w1c
                b1r = bcast(b1v, r)
                h0 = jnp.maximum(allsum(s0) + b1r, 0.0)
                h1 = jnp.maximum(allsum(s1) + b1r, 0.0)
                glob = glob + (m0 * h0 + m1 * h1) * w2t_v[r, :]
            e = jnp.maximum(glob, 0.0)
            q = jnp.minimum(e.astype(jnp.int32), _NK - 2)
            qf = q.astype(jnp.float32)
            w_big = e - qf
            w_small = (qf + 1.0) - e
            for i in range(_NK):
                ms = (1 - jnp.minimum(jnp.abs(q - i), 1)).astype(jnp.float32)
                mb = (1 - jnp.minimum(jnp.abs(q - (i - 1)),
                                      1)).astype(jnp.float32)
                av[i, :] = ms * w_small + mb * w_big
            nv[...] = q + 1
            for i in range(_NK):
                pltpu.sync_copy(av.at[i],
                                alpha_h.at[pl.ds(i * bdim * cdim + flat,
                                                 lanes)])
            pltpu.sync_copy(nv, nlev_h.at[pl.ds(flat, lanes)])

    return sc_router


def _router_body(content_ref, w1_ref, b1_ref, w2_ref, b2_ref,
                 alpha_ref, nlev_ref):
    content = content_ref[...]
    hidden = jnp.maximum(
        jax.lax.dot_general(content, w1_ref[...],
                            (((1,), (1,)), ((), ())),
                            preferred_element_type=jnp.float32)
        + b1_ref[0, :][None, :], 0.0)
    glob = jax.lax.dot_general(hidden, w2_ref[...],
                               (((1,), (1,)), ((), ())),
                               preferred_element_type=jnp.float32) \
        + b2_ref[0, :][None, :]
    e = jnp.maximum(glob, 0.0)
    q_s = jnp.clip(jnp.floor(e), 0.0, float(_NK - 2))
    w_big = e - q_s
    w_small = (q_s + 1.0) - e
    for i in range(_NK):
        fi = float(i)
        alpha = jnp.where(q_s == fi, w_small, 0.0) \
            + jnp.where(q_s == fi - 1.0, w_big, 0.0)
        alpha_ref[i, :, :] = alpha
    nlev_ref[...] = (q_s + 1.0).astype(jnp.int32)


def _pool_body(alpha_ref, nlev_ref, x_ref, o_ref, a_ref, b_ref, *, h, w, c,
               cb):
    g = pl.program_id(0)

    hp = h + 2 * _PADR
    wp = w + 2 * _PADL

    sdt = a_ref.dtype
    a_ref[:, 0:_PADR, :] = jnp.full((cb, _PADR, wp), _NEG, sdt)
    a_ref[:, _PADR + h:, :] = jnp.full((cb, _PADR, wp), _NEG, sdt)
    a_ref[:, :, _PADL - 8:_PADL] = jnp.full((cb, hp, 8), _NEG, sdt)
    a_ref[:, :, _PADL + w:_PADL + w + 8] = jnp.full((cb, hp, 8), _NEG, sdt)
    x = x_ref[...]
    a_ref[:, _PADR:_PADR + h, _PADL:_PADL + w] = x.astype(sdt)

    nlevs = []
    nlev_all = 0
    for j in range(cb):
        bc = g * cb + j
        bi = bc // c
        ci = bc % c
        a0 = alpha_ref[0, bi, ci]
        o_ref[j] = (1.0 + a0) * x[j]
        nl = nlev_ref[bi, ci]
        nlevs.append((nl, bi, ci))
        nlev_all = jnp.maximum(nlev_all, nl)

    for i in range(1, _NK):
        r = (_NK - 1) - i

        @pl.when(i <= nlev_all)
        def _():
            lo = _PADR - r - 1
            hi = _PADR + h + r + 1
            rl = _PADR - r
            rh = _PADR + h + r
            cl = _PADL - r
            ch = _PADL + w + r
            v = a_ref[...]
            b_ref[:, lo:hi, cl:ch] = jnp.maximum(
                jnp.maximum(v[:, lo:hi, cl - 1:ch - 1], v[:, lo:hi, cl:ch]),
                v[:, lo:hi, cl + 1:ch + 1])
            u = b_ref[...]
            a_ref[:, rl:rh, cl:ch] = jnp.maximum(
                jnp.maximum(u[:, lo:rh - 1, cl:ch], u[:, rl:rh, cl:ch]),
                u[:, rl + 1:hi, cl:ch])

            for j in range(cb):
                nl, bi, ci = nlevs[j]

                @pl.when((i >= nl - 1) & (i <= nl))
                def _():
                    ai = alpha_ref[i, bi, ci]
                    p = a_ref[j, _PADR:_PADR + h,
                              _PADL:_PADL + w].astype(jnp.float32)
                    o_ref[j] = o_ref[j] + ai * p


def kernel(x, W1, b1, W2, b2):
    b, c, h, w = x.shape
    cr = W1.shape[0]
    bc = b * c
    xf = x.reshape(bc, h, w)

    cb = 16
    assert bc % cb == 0
    sums = pl.pallas_call(
        _means_body,
        grid=(bc // cb,),
        in_specs=[pl.BlockSpec((cb, h, w), lambda i: (i, 0, 0))],
        out_specs=pl.BlockSpec((1, 1, cb), lambda i: (i, 0, 0)),
        out_shape=jax.ShapeDtypeStruct((bc // cb, 1, cb), jnp.float32),
    )(xf)
    content = sums.reshape(b, c) * (1.0 / (h * w))

    if b == 2 and c % 16 == 0 and cr <= 16 and (b * c) // 16 <= 32:
        b1p = jnp.zeros((16,), jnp.float32).at[:cr].set(b1)
        alpha_f, nlev_f = _make_sc_router(b, c, cr)(
            content, W1, b1p, W2.T.reshape(-1), b2)
        alpha = alpha_f.reshape(_NK, b, c)
        nlev = nlev_f.reshape(b, c)
    else:
        alpha, nlev = pl.pallas_call(
            _router_body,
            out_shape=(jax.ShapeDtypeStruct((_NK, b, c), jnp.float32),
                       jax.ShapeDtypeStruct((b, c), jnp.int32)),
        )(content, W1, b1.reshape(1, cr), W2, b2.reshape(1, c))

    hp, wp = h + 2 * _PADR, w + 2 * _PADL
    pb = 2
    out = pl.pallas_call(
        functools.partial(_pool_body, h=h, w=w, c=c, cb=pb),
        grid=(bc // pb,),
        in_specs=[
            pl.BlockSpec(memory_space=pltpu.SMEM),
            pl.BlockSpec(memory_space=pltpu.SMEM),
            pl.BlockSpec((pb, h, w), lambda i: (i, 0, 0)),
        ],
        out_specs=pl.BlockSpec((pb, h, w), lambda i: (i, 0, 0)),
        out_shape=jax.ShapeDtypeStruct((bc, h, w), jnp.float32),
        scratch_shapes=[pltpu.VMEM((pb, hp, wp), jnp.bfloat16),
                        pltpu.VMEM((pb, hp, wp), jnp.bfloat16)],
    )(alpha, nlev, xf)
    return out.reshape(b, c, h, w)


# speedup vs baseline: 1.0068x; 1.0068x over previous
import functools

import jax
import jax.numpy as jnp
from jax.experimental import pallas as pl
from jax.experimental.pallas import tpu as pltpu
from jax.experimental.pallas import tpu_sc as plsc

_KS = (1, 3, 5, 7, 9, 11, 13)
_NK = len(_KS)
_PADR = 8
_PADL = 8
_NEG = -jnp.inf


def _means_body(x_ref, o_ref):
    s = jnp.sum(x_ref[...], axis=(1, 2))
    o_ref[0, 0, :] = s


def _make_sc_router(bdim, cdim, crdim):
    info = plsc.get_sparse_core_info()
    nw = info.num_cores * info.num_subcores
    lanes = info.num_lanes
    ngroups = bdim * cdim // lanes
    gper_b = cdim // lanes
    assert ngroups <= nw and cdim % lanes == 0 and crdim <= lanes
    mesh = plsc.VectorSubcoreMesh(core_axis_name="c", subcore_axis_name="s")

    @functools.partial(
        pl.kernel, mesh=mesh,
        out_type=(jax.ShapeDtypeStruct((ngroups, _NK, lanes), jnp.float32),
                  jax.ShapeDtypeStruct((bdim * cdim,), jnp.int32)),
        scratch_types=[
            pltpu.VMEM((bdim, cdim), jnp.float32),
            pltpu.VMEM((crdim, cdim), jnp.float32),
            pltpu.VMEM((lanes,), jnp.float32),
            pltpu.VMEM((crdim, lanes), jnp.float32),
            pltpu.VMEM((lanes,), jnp.float32),
            pltpu.VMEM((_NK, lanes), jnp.float32),
            pltpu.VMEM((lanes,), jnp.int32),
        ],
    )
    def sc_router(content_h, w1_h, b1_h, w2g_h, b2_h, alpha_h, nlev_h,
                  content_v, w1_v, b1_v, w2t_v, b2_v, av, nv):
        wid = (jax.lax.axis_index("s") * info.num_cores
               + jax.lax.axis_index("c"))

        @pl.when(wid < ngroups)
        def _():
            g2 = wid % gper_b
            flat = wid * lanes
            pltpu.sync_copy(content_h, content_v)
            pltpu.sync_copy(w1_h, w1_v)
            pltpu.sync_copy(b1_h, b1_v)
            pltpu.sync_copy(w2g_h.at[g2], w2t_v)
            pltpu.sync_copy(b2_h.at[g2], b2_v)

            def shuf(v, idx):
                return jax.lax.gather(
                    v, idx[:, None],
                    jax.lax.GatherDimensionNumbers(
                        offset_dims=(), collapsed_slice_dims=(0,),
                        start_index_map=(0,)),
                    (1,), mode=jax.lax.GatherScatterMode.PROMISE_IN_BOUNDS)

            def bcast(v, i):
                return shuf(v, jnp.full((lanes,), i, jnp.int32))

            lane = jax.lax.iota(jnp.int32, lanes)

            def allsum(v):
                for s in (1, 2, 4, 8):
                    v = v + shuf(v, lane ^ s)
                return v

            vwid = jnp.zeros((lanes,), jnp.int32) + wid
            m0 = jnp.minimum(jnp.maximum(gper_b - vwid, 0),
                             1).astype(jnp.float32)
            m1 = 1.0 - m0
            b1v = b1_v[...]
            glob = b2_v[...]
            for r in range(crdim):
                s0 = jnp.zeros((lanes,), jnp.float32)
                s1 = jnp.zeros((lanes,), jnp.float32)
                for t in range(cdim // lanes):
                    w1c = w1_v[r, pl.ds(t * lanes, lanes)]
                    s0 = s0 + content_v[0, pl.ds(t * lanes, lanes)] * w1c
                    s1 = s1 + content_v[1, pl.ds(t * lanes, lanes)] * w1c
                b1r = bcast(b1v, r)
                h0 = jnp.maximum(allsum(s0) + b1r, 0.0)
                h1 = jnp.maximum(allsum(s1) + b1r, 0.0)
                glob = glob + (m0 * h0 + m1 * h1) * w2t_v[r, :]
            e = jnp.maximum(glob, 0.0)
            q = jnp.minimum(e.astype(jnp.int32), _NK - 2)
            qf = q.astype(jnp.float32)
            w_big = e - qf
            w_small = (qf + 1.0) - e
            for i in range(_NK):
                ms = (1 - jnp.minimum(jnp.abs(q - i), 1)).astype(jnp.float32)
                mb = (1 - jnp.minimum(jnp.abs(q - (i - 1)),
                                      1)).astype(jnp.float32)
                av[i, :] = ms * w_small + mb * w_big
            nv[...] = q + 1
            pltpu.sync_copy(av, alpha_h.at[wid])
            pltpu.sync_copy(nv, nlev_h.at[pl.ds(flat, lanes)])

    return sc_router


def _router_body(content_ref, w1_ref, b1_ref, w2_ref, b2_ref,
                 alpha_ref, nlev_ref):
    content = content_ref[...]
    hidden = jnp.maximum(
        jax.lax.dot_general(content, w1_ref[...],
                            (((1,), (1,)), ((), ())),
                            preferred_element_type=jnp.float32)
        + b1_ref[0, :][None, :], 0.0)
    glob = jax.lax.dot_general(hidden, w2_ref[...],
                               (((1,), (1,)), ((), ())),
                               preferred_element_type=jnp.float32) \
        + b2_ref[0, :][None, :]
    e = jnp.maximum(glob, 0.0)
    q_s = jnp.clip(jnp.floor(e), 0.0, float(_NK - 2))
    w_big = e - q_s
    w_small = (q_s + 1.0) - e
    for i in range(_NK):
        fi = float(i)
        alpha = jnp.where(q_s == fi, w_small, 0.0) \
            + jnp.where(q_s == fi - 1.0, w_big, 0.0)
        alpha_ref[i, :, :] = alpha
    nlev_ref[...] = (q_s + 1.0).astype(jnp.int32)


def _pool_body(alpha_ref, nlev_ref, x_ref, o_ref, a_ref, b_ref, *, h, w, c,
               cb):
    g = pl.program_id(0)

    hp = h + 2 * _PADR
    wp = w + 2 * _PADL

    sdt = a_ref.dtype
    a_ref[:, 0:_PADR, :] = jnp.full((cb, _PADR, wp), _NEG, sdt)
    a_ref[:, _PADR + h:, :] = jnp.full((cb, _PADR, wp), _NEG, sdt)
    a_ref[:, :, _PADL - 8:_PADL] = jnp.full((cb, hp, 8), _NEG, sdt)
    a_ref[:, :, _PADL + w:_PADL + w + 8] = jnp.full((cb, hp, 8), _NEG, sdt)
    x = x_ref[...]
    a_ref[:, _PADR:_PADR + h, _PADL:_PADL + w] = x.astype(sdt)

    nlevs = []
    nlev_all = 0
    for j in range(cb):
        bc = g * cb + j
        bi = bc // c
        ci = bc % c
        a0 = alpha_ref[0, bi, ci]
        o_ref[j] = (1.0 + a0) * x[j]
        nl = nlev_ref[bi, ci]
        nlevs.append((nl, bi, ci))
        nlev_all = jnp.maximum(nlev_all, nl)

    for i in range(1, _NK):
        r = (_NK - 1) - i

        @pl.when(i <= nlev_all)
        def _():
            lo = _PADR - r - 1
            hi = _PADR + h + r + 1
            rl = _PADR - r
            rh = _PADR + h + r
            cl = _PADL - r
            ch = _PADL + w + r
            v = a_ref[...]
            b_ref[:, lo:hi, cl:ch] = jnp.maximum(
                jnp.maximum(v[:, lo:hi, cl - 1:ch - 1], v[:, lo:hi, cl:ch]),
                v[:, lo:hi, cl + 1:ch + 1])
            u = b_ref[...]
            a_ref[:, rl:rh, cl:ch] = jnp.maximum(
                jnp.maximum(u[:, lo:rh - 1, cl:ch], u[:, rl:rh, cl:ch]),
                u[:, rl + 1:hi, cl:ch])

            for j in range(cb):
                nl, bi, ci = nlevs[j]

                @pl.when((i >= nl - 1) & (i <= nl))
                def _():
                    ai = alpha_ref[i, bi, ci]
                    p = a_ref[j, _PADR:_PADR + h,
                              _PADL:_PADL + w].astype(jnp.float32)
                    o_ref[j] = o_ref[j] + ai * p


def kernel(x, W1, b1, W2, b2):
    b, c, h, w = x.shape
    cr = W1.shape[0]
    bc = b * c
    xf = x.reshape(bc, h, w)

    cb = 16
    assert bc % cb == 0
    sums = pl.pallas_call(
        _means_body,
        grid=(bc // cb,),
        in_specs=[pl.BlockSpec((cb, h, w), lambda i: (i, 0, 0))],
        out_specs=pl.BlockSpec((1, 1, cb), lambda i: (i, 0, 0)),
        out_shape=jax.ShapeDtypeStruct((bc // cb, 1, cb), jnp.float32),
    )(xf)
    content = sums.reshape(b, c) * (1.0 / (h * w))

    if b == 2 and c % 16 == 0 and cr <= 16 and (b * c) // 16 <= 32:
        b1p = jnp.zeros((16,), jnp.float32).at[:cr].set(b1)
        w2g = W2.T.reshape(cr, c // 16, 16).transpose(1, 0, 2)
        alpha_f, nlev_f = _make_sc_router(b, c, cr)(
            content, W1, b1p, w2g, b2.reshape(c // 16, 16))
        alpha = alpha_f.transpose(1, 0, 2).reshape(_NK, b, c)
        nlev = nlev_f.reshape(b, c)
    else:
        alpha, nlev = pl.pallas_call(
            _router_body,
            out_shape=(jax.ShapeDtypeStruct((_NK, b, c), jnp.float32),
                       jax.ShapeDtypeStruct((b, c), jnp.int32)),
        )(content, W1, b1.reshape(1, cr), W2, b2.reshape(1, c))

    hp, wp = h + 2 * _PADR, w + 2 * _PADL
    pb = 2
    out = pl.pallas_call(
        functools.partial(_pool_body, h=h, w=w, c=c, cb=pb),
        grid=(bc // pb,),
        in_specs=[
            pl.BlockSpec(memory_space=pltpu.SMEM),
            pl.BlockSpec(memory_space=pltpu.SMEM),
            pl.BlockSpec((pb, h, w), lambda i: (i, 0, 0)),
        ],
        out_specs=pl.BlockSpec((pb, h, w), lambda i: (i, 0, 0)),
        out_shape=jax.ShapeDtypeStruct((bc, h, w), jnp.float32),
        scratch_shapes=[pltpu.VMEM((pb, hp, wp), jnp.bfloat16),
                        pltpu.VMEM((pb, hp, wp), jnp.bfloat16)],
    )(alpha, nlev, xf)
    return out.reshape(b, c, h, w)
